# speedup vs baseline: 1.0660x; 1.0660x over previous
import functools

import jax
import jax.numpy as jnp
from jax.experimental import pallas as pl
from jax.experimental.pallas import tpu as pltpu

N = 10000
E = 320000
D = 128

_BLK = 1000


def _scale_matmul_body(x_ref, w_ref, deg_ref, o_ref):
    h = jnp.dot(x_ref[...], w_ref[...], preferred_element_type=jnp.float32)
    norm = jax.lax.rsqrt(jnp.clip(deg_ref[...], 1.0, None))
    o_ref[...] = h * norm


def _tc_scale_matmul(x, W, deg):
    return pl.pallas_call(
        _scale_matmul_body,
        grid=(N // _BLK,),
        in_specs=[
            pl.BlockSpec((_BLK, D), lambda i: (i, 0)),
            pl.BlockSpec((D, D), lambda i: (0, 0)),
            pl.BlockSpec((_BLK, 1), lambda i: (i, 0)),
        ],
        out_specs=pl.BlockSpec((_BLK, D), lambda i: (i, 0)),
        out_shape=jax.ShapeDtypeStruct((N, D), jnp.float32),
    )(x, W, deg)


def _post_body(p0_ref, p1_ref, deg_ref, b_ref, o_ref):
    norm = jax.lax.rsqrt(jnp.clip(deg_ref[...], 1.0, None))
    o_ref[...] = jnp.maximum((p0_ref[...] + p1_ref[...]) * norm + b_ref[...], 0.0)


def _tc_post(p0, p1, deg, b):
    return pl.pallas_call(
        _post_body,
        grid=(N // _BLK,),
        in_specs=[
            pl.BlockSpec((_BLK, D), lambda i: (i, 0)),
            pl.BlockSpec((_BLK, D), lambda i: (i, 0)),
            pl.BlockSpec((_BLK, 1), lambda i: (i, 0)),
            pl.BlockSpec((1, D), lambda i: (0, 0)),
        ],
        out_specs=pl.BlockSpec((_BLK, D), lambda i: (i, 0)),
        out_shape=jax.ShapeDtypeStruct((N, D), jnp.float32),
    )(p0, p1, deg, b)


def _post_matmul_body(p0_ref, p1_ref, degi_ref, b_ref, w_ref, dego_ref, o_ref):
    normi = jax.lax.rsqrt(jnp.clip(degi_ref[...], 1.0, None))
    h = jnp.maximum((p0_ref[...] + p1_ref[...]) * normi + b_ref[...], 0.0)
    normo = jax.lax.rsqrt(jnp.clip(dego_ref[...], 1.0, None))
    o_ref[...] = jnp.dot(h, w_ref[...], preferred_element_type=jnp.float32) * normo


def _tc_post_matmul(p0, p1, deg_in, b, W, deg_out):
    return pl.pallas_call(
        _post_matmul_body,
        grid=(N // _BLK,),
        in_specs=[
            pl.BlockSpec((_BLK, D), lambda i: (i, 0)),
            pl.BlockSpec((_BLK, D), lambda i: (i, 0)),
            pl.BlockSpec((_BLK, 1), lambda i: (i, 0)),
            pl.BlockSpec((1, D), lambda i: (0, 0)),
            pl.BlockSpec((D, D), lambda i: (0, 0)),
            pl.BlockSpec((_BLK, 1), lambda i: (i, 0)),
        ],
        out_specs=pl.BlockSpec((_BLK, D), lambda i: (i, 0)),
        out_shape=jax.ShapeDtypeStruct((N, D), jnp.float32),
    )(p0, p1, deg_in, b, W, deg_out)


def _degrees(idx):
    ones = jnp.ones((E,), dtype=jnp.float32)
    return jax.ops.segment_sum(ones, idx, num_segments=N)[:, None]


@jax.jit
def kernel(x, edge_index0, edge_index1, score_edge_index, W1, b1, W2, b2):
    src0, dst0 = edge_index0[0], edge_index0[1]
    src1, dst1 = edge_index1[0], edge_index1[1]

    deg_out0 = _degrees(src0)
    deg_in0 = _degrees(dst0)
    deg_out1 = _degrees(src1)
    deg_in1 = _degrees(dst1)

    hs1 = _tc_scale_matmul(x, W1, deg_out0)
    agg1 = jax.ops.segment_sum(hs1[src0], dst0, num_segments=N)
    half = jnp.zeros_like(agg1)
    hs2 = _tc_post_matmul(agg1, half, deg_in0, b1[None, :], W2, deg_out1)
    agg2 = jax.ops.segment_sum(hs2[src1], dst1, num_segments=N)
    h2 = _tc_post(agg2, half, deg_in1, b2[None, :])

    u = score_edge_index[0]
    v = score_edge_index[1]
    score = jnp.sum(h2[u] * h2[v], axis=-1, keepdims=True)
    return score


# speedup vs baseline: 2.5599x; 2.4015x over previous
import functools

import jax
import jax.numpy as jnp
from jax import lax
from jax.experimental import pallas as pl
from jax.experimental.pallas import tpu as pltpu
from jax.experimental.pallas import tpu_sc as plsc

N = 10000
E = 320000
D = 128

_NC = 2
_NS = 16
_NW = _NC * _NS
_EPW = E // _NW
_CW = 125
_CH = _EPW // _CW
_NPAD = 10112
_RPT = _NPAD // _NS
_HPW = E // _NS
_HCH = _HPW // _CW

_BLK = 1000


def _scale_matmul_body(x_ref, w_ref, deg_ref, o_ref):
    h = jnp.dot(x_ref[...], w_ref[...], preferred_element_type=jnp.float32)
    norm = jax.lax.rsqrt(jnp.clip(deg_ref[...], 1.0, None))
    o_ref[...] = h * norm


def _tc_scale_matmul(x, W, deg):
    return pl.pallas_call(
        _scale_matmul_body,
        grid=(N // _BLK,),
        in_specs=[
            pl.BlockSpec((_BLK, D), lambda i: (i, 0)),
            pl.BlockSpec((D, D), lambda i: (0, 0)),
            pl.BlockSpec((_BLK, 1), lambda i: (i, 0)),
        ],
        out_specs=pl.BlockSpec((_BLK, D), lambda i: (i, 0)),
        out_shape=jax.ShapeDtypeStruct((N, D), jnp.float32),
    )(x, W, deg)


def _post_body(p0_ref, p1_ref, deg_ref, b_ref, o_ref):
    norm = jax.lax.rsqrt(jnp.clip(deg_ref[...], 1.0, None))
    o_ref[...] = jnp.maximum((p0_ref[...] + p1_ref[...]) * norm + b_ref[...], 0.0)


def _tc_post(p0, p1, deg, b):
    return pl.pallas_call(
        _post_body,
        grid=(N // _BLK,),
        in_specs=[
            pl.BlockSpec((_BLK, D), lambda i: (i, 0)),
            pl.BlockSpec((_BLK, D), lambda i: (i, 0)),
            pl.BlockSpec((_BLK, 1), lambda i: (i, 0)),
            pl.BlockSpec((1, D), lambda i: (0, 0)),
        ],
        out_specs=pl.BlockSpec((_BLK, D), lambda i: (i, 0)),
        out_shape=jax.ShapeDtypeStruct((N, D), jnp.float32),
    )(p0, p1, deg, b)


def _post_matmul_body(p0_ref, p1_ref, degi_ref, b_ref, w_ref, dego_ref, o_ref):
    normi = jax.lax.rsqrt(jnp.clip(degi_ref[...], 1.0, None))
    h = jnp.maximum((p0_ref[...] + p1_ref[...]) * normi + b_ref[...], 0.0)
    normo = jax.lax.rsqrt(jnp.clip(dego_ref[...], 1.0, None))
    o_ref[...] = jnp.dot(h, w_ref[...], preferred_element_type=jnp.float32) * normo


def _tc_post_matmul(p0, p1, deg_in, b, W, deg_out):
    return pl.pallas_call(
        _post_matmul_body,
        grid=(N // _BLK,),
        in_specs=[
            pl.BlockSpec((_BLK, D), lambda i: (i, 0)),
            pl.BlockSpec((_BLK, D), lambda i: (i, 0)),
            pl.BlockSpec((_BLK, 1), lambda i: (i, 0)),
            pl.BlockSpec((1, D), lambda i: (0, 0)),
            pl.BlockSpec((D, D), lambda i: (0, 0)),
            pl.BlockSpec((_BLK, 1), lambda i: (i, 0)),
        ],
        out_specs=pl.BlockSpec((_BLK, D), lambda i: (i, 0)),
        out_shape=jax.ShapeDtypeStruct((N, D), jnp.float32),
    )(p0, p1, deg_in, b, W, deg_out)


_MESH = plsc.VectorSubcoreMesh(core_axis_name="c", subcore_axis_name="s")


def _zero_rows(buf, nrows):
    ncol = buf.shape[1] // 16
    zeros = jnp.zeros((16,), jnp.float32)

    def row(i, carry):
        for k in range(ncol):
            buf[i, pl.ds(16 * k, 16)] = zeros
        return carry

    lax.fori_loop(0, nrows, row, 0)


def _zero_acc(zb, acc, base):
    _zero_rows(zb, 8)

    def blk(i, carry):
        pltpu.sync_copy(zb, acc.at[pl.ds(base + i * 8, 8)])
        return carry

    lax.fori_loop(0, _RPT // 8, blk, 0)


def _conv_body(hs_hbm, src_hbm, dst_hbm, out_hbm,
               idx_s, idx_d, rows, zbuf, acc, sem):
    c = lax.axis_index("c")
    s = lax.axis_index("s")
    pltpu.sync_copy(src_hbm.at[c, s], idx_s)
    pltpu.sync_copy(dst_hbm.at[c, s], idx_d)
    _zero_acc(zbuf, acc, s * _RPT)
    plsc.subcore_barrier()

    def chunk(j, carry):
        pltpu.async_copy(hs_hbm.at[idx_s.at[j]], rows, sem).wait()
        pltpu.sync_copy(rows, acc.at[idx_d.at[j]], add=True)
        return carry

    lax.fori_loop(0, _CH, chunk, 0)
    plsc.subcore_barrier()
    pltpu.sync_copy(acc.at[pl.ds(s * _RPT, _RPT)],
                    out_hbm.at[c, pl.ds(s * _RPT, _RPT)])


_conv = pl.kernel(
    _conv_body,
    out_type=jax.ShapeDtypeStruct((_NC, _NPAD, D), jnp.float32),
    mesh=_MESH,
    scratch_types=[
        pltpu.VMEM((_CH, _CW), jnp.int32),
        pltpu.VMEM((_CH, _CW), jnp.int32),
        pltpu.VMEM((_CW, D), jnp.float32),
        pltpu.VMEM((8, D), jnp.float32),
        pltpu.VMEM_SHARED((_NPAD, D), jnp.float32),
        pltpu.SemaphoreType.DMA,
    ],
)


def _hist_body(idx_hbm, out_hbm, idx_s, idx_d, ones, zb, acc_s, acc_d, sem):
    c = lax.axis_index("c")
    s = lax.axis_index("s")
    pltpu.sync_copy(idx_hbm.at[c, 0, s], idx_s)
    pltpu.sync_copy(idx_hbm.at[c, 1, s], idx_d)
    one = jnp.ones((16,), jnp.float32)

    def orow(i, carry):
        ones[i, pl.ds(0, 16)] = one
        return carry

    lax.fori_loop(0, _CW, orow, 0)
    _zero_acc(zb, acc_s, s * _RPT)
    _zero_acc(zb, acc_d, s * _RPT)
    plsc.subcore_barrier()

    def chunk(j, carry):
        pltpu.sync_copy(ones, acc_s.at[idx_s.at[j]], add=True)
        pltpu.sync_copy(ones, acc_d.at[idx_d.at[j]], add=True)
        return carry

    lax.fori_loop(0, _HCH, chunk, 0)
    plsc.subcore_barrier()
    pltpu.sync_copy(acc_s.at[pl.ds(s * _RPT, _RPT)],
                    out_hbm.at[c, 0, pl.ds(s * _RPT, _RPT)])
    pltpu.sync_copy(acc_d.at[pl.ds(s * _RPT, _RPT)],
                    out_hbm.at[c, 1, pl.ds(s * _RPT, _RPT)])


_hist = pl.kernel(
    _hist_body,
    out_type=jax.ShapeDtypeStruct((_NC, 2, _NPAD, 16), jnp.float32),
    mesh=_MESH,
    scratch_types=[
        pltpu.VMEM((_HCH, _CW), jnp.int32),
        pltpu.VMEM((_HCH, _CW), jnp.int32),
        pltpu.VMEM((_CW, 16), jnp.float32),
        pltpu.VMEM((8, 16), jnp.float32),
        pltpu.VMEM_SHARED((_NPAD, 16), jnp.float32),
        pltpu.VMEM_SHARED((_NPAD, 16), jnp.float32),
        pltpu.SemaphoreType.DMA,
    ],
)


def _score_body(h_hbm, u_hbm, v_hbm, out_hbm,
                idx_u, idx_v, ubuf, vbuf, obuf, semu, semv):
    c = lax.axis_index("c")
    s = lax.axis_index("s")
    pltpu.sync_copy(u_hbm.at[c, s], idx_u)
    pltpu.sync_copy(v_hbm.at[c, s], idx_v)

    def chunk(j, carry):
        cu = pltpu.async_copy(h_hbm.at[idx_u.at[j]], ubuf, semu)
        cv = pltpu.async_copy(h_hbm.at[idx_v.at[j]], vbuf, semv)
        cu.wait()
        cv.wait()

        def row(r, carry2):
            acc = ubuf[r, pl.ds(0, 16)] * vbuf[r, pl.ds(0, 16)]
            for k in range(1, D // 16):
                acc = acc + ubuf[r, pl.ds(16 * k, 16)] * vbuf[r, pl.ds(16 * k, 16)]
            obuf[r, pl.ds(0, 16)] = acc
            return carry2

        lax.fori_loop(0, _CW, row, 0)
        pltpu.sync_copy(obuf, out_hbm.at[c, s, j])
        return carry

    lax.fori_loop(0, _CH, chunk, 0)


_score = pl.kernel(
    _score_body,
    out_type=jax.ShapeDtypeStruct((_NC, _NS, _CH, _CW, 16), jnp.float32),
    mesh=_MESH,
    scratch_types=[
        pltpu.VMEM((_CH, _CW), jnp.int32),
        pltpu.VMEM((_CH, _CW), jnp.int32),
        pltpu.VMEM((_CW, D), jnp.float32),
        pltpu.VMEM((_CW, D), jnp.float32),
        pltpu.VMEM((_CW, 16), jnp.float32),
        pltpu.SemaphoreType.DMA,
        pltpu.SemaphoreType.DMA,
    ],
)


def _reduce_body(p_ref, o_ref):
    o_ref[...] = jnp.sum(p_ref[...], axis=1, keepdims=True)


def _tc_reduce16(p):
    blk = 2000
    return pl.pallas_call(
        _reduce_body,
        grid=(E // blk,),
        in_specs=[pl.BlockSpec((blk, 16), lambda i: (i, 0))],
        out_specs=pl.BlockSpec((blk, 1), lambda i: (i, 0)),
        out_shape=jax.ShapeDtypeStruct((E, 1), jnp.float32),
    )(p)


@jax.jit
def kernel(x, edge_index0, edge_index1, score_edge_index, W1, b1, W2, b2):
    ones = jnp.ones((E,), dtype=jnp.float32)
    deg_out0 = jax.ops.segment_sum(ones, edge_index0[0], num_segments=N)[:, None]
    deg_in0 = jax.ops.segment_sum(ones, edge_index0[1], num_segments=N)[:, None]
    deg_out1 = jax.ops.segment_sum(ones, edge_index1[0], num_segments=N)[:, None]
    deg_in1 = jax.ops.segment_sum(ones, edge_index1[1], num_segments=N)[:, None]

    src0 = edge_index0[0].reshape(_NC, _NS, _CH, _CW)
    dst0 = edge_index0[1].reshape(_NC, _NS, _CH, _CW)
    src1 = edge_index1[0].reshape(_NC, _NS, _CH, _CW)
    dst1 = edge_index1[1].reshape(_NC, _NS, _CH, _CW)

    hs1 = _tc_scale_matmul(x, W1, deg_out0)
    p1 = _conv(hs1, src0, dst0)
    hs2 = _tc_post_matmul(p1[0, :N], p1[1, :N], deg_in0, b1[None, :],
                          W2, deg_out1)
    p2 = _conv(hs2, src1, dst1)
    h2 = _tc_post(p2[0, :N], p2[1, :N], deg_in1, b2[None, :])

    u = score_edge_index[0].reshape(_NC, _NS, _CH, _CW)
    v = score_edge_index[1].reshape(_NC, _NS, _CH, _CW)
    partial = _score(h2, u, v).reshape(E, 16)
    return _tc_reduce16(partial)


# speedup vs baseline: 4.4938x; 1.7555x over previous
import functools

import jax
import jax.numpy as jnp
from jax import lax
from jax.experimental import pallas as pl
from jax.experimental.pallas import tpu as pltpu
from jax.experimental.pallas import tpu_sc as plsc

N = 10000
E = 320000
D = 128

_NC = 2
_NS = 16
_NW = _NC * _NS
_EPW = E // _NW
_CW = 125
_CH = _EPW // _CW
_NPAD = 10112
_RPT = _NPAD // _NS
_HPW = E // _NS
_HCH = _HPW // _CW

_BLK = 1000


def _scale_matmul_body(x_ref, w_ref, deg_ref, o_ref):
    h = jnp.dot(x_ref[...], w_ref[...], preferred_element_type=jnp.float32)
    norm = jax.lax.rsqrt(jnp.clip(deg_ref[...], 1.0, None))
    o_ref[...] = h * norm


def _tc_scale_matmul(x, W, deg):
    return pl.pallas_call(
        _scale_matmul_body,
        grid=(N // _BLK,),
        in_specs=[
            pl.BlockSpec((_BLK, D), lambda i: (i, 0)),
            pl.BlockSpec((D, D), lambda i: (0, 0)),
            pl.BlockSpec((_BLK, 1), lambda i: (i, 0)),
        ],
        out_specs=pl.BlockSpec((_BLK, D), lambda i: (i, 0)),
        out_shape=jax.ShapeDtypeStruct((N, D), jnp.float32),
    )(x, W, deg)


def _post_body(p0_ref, p1_ref, deg_ref, b_ref, o_ref):
    norm = jax.lax.rsqrt(jnp.clip(deg_ref[...], 1.0, None))
    o_ref[...] = jnp.maximum((p0_ref[...] + p1_ref[...]) * norm + b_ref[...], 0.0)


def _tc_post(p0, p1, deg, b):
    return pl.pallas_call(
        _post_body,
        grid=(N // _BLK,),
        in_specs=[
            pl.BlockSpec((_BLK, D), lambda i: (i, 0)),
            pl.BlockSpec((_BLK, D), lambda i: (i, 0)),
            pl.BlockSpec((_BLK, 1), lambda i: (i, 0)),
            pl.BlockSpec((1, D), lambda i: (0, 0)),
        ],
        out_specs=pl.BlockSpec((_BLK, D), lambda i: (i, 0)),
        out_shape=jax.ShapeDtypeStruct((N, D), jnp.float32),
    )(p0, p1, deg, b)


def _post_matmul_body(p0_ref, p1_ref, degi_ref, b_ref, w_ref, dego_ref, o_ref):
    normi = jax.lax.rsqrt(jnp.clip(degi_ref[...], 1.0, None))
    h = jnp.maximum((p0_ref[...] + p1_ref[...]) * normi + b_ref[...], 0.0)
    normo = jax.lax.rsqrt(jnp.clip(dego_ref[...], 1.0, None))
    o_ref[...] = jnp.dot(h, w_ref[...], preferred_element_type=jnp.float32) * normo


def _tc_post_matmul(p0, p1, deg_in, b, W, deg_out):
    return pl.pallas_call(
        _post_matmul_body,
        grid=(N // _BLK,),
        in_specs=[
            pl.BlockSpec((_BLK, D), lambda i: (i, 0)),
            pl.BlockSpec((_BLK, D), lambda i: (i, 0)),
            pl.BlockSpec((_BLK, 1), lambda i: (i, 0)),
            pl.BlockSpec((1, D), lambda i: (0, 0)),
            pl.BlockSpec((D, D), lambda i: (0, 0)),
            pl.BlockSpec((_BLK, 1), lambda i: (i, 0)),
        ],
        out_specs=pl.BlockSpec((_BLK, D), lambda i: (i, 0)),
        out_shape=jax.ShapeDtypeStruct((N, D), jnp.float32),
    )(p0, p1, deg_in, b, W, deg_out)


_MESH = plsc.VectorSubcoreMesh(core_axis_name="c", subcore_axis_name="s")


def _zero_rows(buf, nrows):
    ncol = buf.shape[1] // 16
    zeros = jnp.zeros((16,), jnp.float32)

    def row(i, carry):
        for k in range(ncol):
            buf[i, pl.ds(16 * k, 16)] = zeros
        return carry

    lax.fori_loop(0, nrows, row, 0)


def _zero_acc(zb, acc, base):
    _zero_rows(zb, 8)

    def blk(i, carry):
        pltpu.sync_copy(zb, acc.at[pl.ds(base + i * 8, 8)])
        return carry

    lax.fori_loop(0, _RPT // 8, blk, 0)


def _conv_body(hs_hbm, src_hbm, dst_hbm, out_hbm,
               idx_s, idx_d, rows, zbuf, acc, sem):
    c = lax.axis_index("c")
    s = lax.axis_index("s")
    pltpu.sync_copy(src_hbm.at[c, s], idx_s)
    pltpu.sync_copy(dst_hbm.at[c, s], idx_d)
    _zero_acc(zbuf, acc, s * _RPT)
    plsc.subcore_barrier()

    def chunk(j, carry):
        pltpu.async_copy(hs_hbm.at[idx_s.at[j]], rows, sem).wait()
        pltpu.sync_copy(rows, acc.at[idx_d.at[j]], add=True)
        return carry

    lax.fori_loop(0, _CH, chunk, 0)
    plsc.subcore_barrier()
    pltpu.sync_copy(acc.at[pl.ds(s * _RPT, _RPT)],
                    out_hbm.at[c, pl.ds(s * _RPT, _RPT)])


_conv = pl.kernel(
    _conv_body,
    out_type=jax.ShapeDtypeStruct((_NC, _NPAD, D), jnp.float32),
    mesh=_MESH,
    scratch_types=[
        pltpu.VMEM((_CH, _CW), jnp.int32),
        pltpu.VMEM((_CH, _CW), jnp.int32),
        pltpu.VMEM((_CW, D), jnp.float32),
        pltpu.VMEM((8, D), jnp.float32),
        pltpu.VMEM_SHARED((_NPAD, D), jnp.float32),
        pltpu.SemaphoreType.DMA,
    ],
)


_HROWS = 80
_HPWP = 20096


def _hist_one(idxbuf, hist):
    iota = lax.iota(jnp.int32, 16)
    pib = jax.lax.GatherScatterMode.PROMISE_IN_BOUNDS

    def step(k, carry):
        for g in range(8):
            idx = idxbuf[k, pl.ds(g * 16, 16)]
            sk, _ = plsc.sort_key_val(idx, idx)
            prev = sk.at[jnp.maximum(iota - 1, 0)].get(mode=pib)
            nxt = sk.at[jnp.minimum(iota + 1, 15)].get(mode=pib)
            head = (iota == 0) | (sk != prev)
            last = (iota == 15) | (sk != nxt)
            start = plsc.cummax(jnp.where(head, iota, 0))
            cnt = (iota - start + 1).astype(jnp.float32)
            plsc.addupdate_scatter(
                hist,
                [lax.shift_right_logical(sk, 7), lax.bitwise_and(sk, 127)],
                cnt, mask=last)
        return carry

    lax.fori_loop(0, _HPWP // D, step, 0)


def _hist_body(idx_hbm, out_hbm, idx_s, idx_d, hist_s, hist_d):
    c = lax.axis_index("c")
    s = lax.axis_index("s")
    pltpu.sync_copy(idx_hbm.at[c, 0, s], idx_s)
    pltpu.sync_copy(idx_hbm.at[c, 1, s], idx_d)
    _zero_rows(hist_s, _HROWS)
    _zero_rows(hist_d, _HROWS)
    _hist_one(idx_s, hist_s)
    _hist_one(idx_d, hist_d)
    pltpu.sync_copy(hist_s, out_hbm.at[c, 0, s])
    pltpu.sync_copy(hist_d, out_hbm.at[c, 1, s])


_hist = pl.kernel(
    _hist_body,
    out_type=jax.ShapeDtypeStruct((_NC, 2, _NS, _HROWS, D), jnp.float32),
    mesh=_MESH,
    scratch_types=[
        pltpu.VMEM((_HPWP // D, D), jnp.int32),
        pltpu.VMEM((_HPWP // D, D), jnp.int32),
        pltpu.VMEM((_HROWS, D), jnp.float32),
        pltpu.VMEM((_HROWS, D), jnp.float32),
    ],
    compiler_params=pltpu.CompilerParams(needs_layout_passes=False),
)


def _sum_tiles_body(p_ref, o_ref):
    o_ref[...] = jnp.sum(p_ref[...], axis=1)


def _tc_sum_tiles(p):
    hb = _HROWS * D
    return pl.pallas_call(
        _sum_tiles_body,
        out_shape=jax.ShapeDtypeStruct((4, hb), jnp.float32),
    )(p)


def _score_body(h_hbm, u_hbm, v_hbm, out_hbm,
                idx_u, idx_v, ubuf, vbuf, obuf, semu, semv):
    c = lax.axis_index("c")
    s = lax.axis_index("s")
    pltpu.sync_copy(u_hbm.at[c, s], idx_u)
    pltpu.sync_copy(v_hbm.at[c, s], idx_v)

    def chunk(j, carry):
        cu = pltpu.async_copy(h_hbm.at[idx_u.at[j]], ubuf, semu)
        cv = pltpu.async_copy(h_hbm.at[idx_v.at[j]], vbuf, semv)
        cu.wait()
        cv.wait()

        def row(r, carry2):
            acc = ubuf[r, pl.ds(0, 16)] * vbuf[r, pl.ds(0, 16)]
            for k in range(1, D // 16):
                acc = acc + ubuf[r, pl.ds(16 * k, 16)] * vbuf[r, pl.ds(16 * k, 16)]
            obuf[r, pl.ds(0, 16)] = acc
            return carry2

        lax.fori_loop(0, _CW, row, 0)
        pltpu.sync_copy(obuf, out_hbm.at[c, s, j])
        return carry

    lax.fori_loop(0, _CH, chunk, 0)


_score = pl.kernel(
    _score_body,
    out_type=jax.ShapeDtypeStruct((_NC, _NS, _CH, _CW, 16), jnp.float32),
    mesh=_MESH,
    scratch_types=[
        pltpu.VMEM((_CH, _CW), jnp.int32),
        pltpu.VMEM((_CH, _CW), jnp.int32),
        pltpu.VMEM((_CW, D), jnp.float32),
        pltpu.VMEM((_CW, D), jnp.float32),
        pltpu.VMEM((_CW, 16), jnp.float32),
        pltpu.SemaphoreType.DMA,
        pltpu.SemaphoreType.DMA,
    ],
)


def _reduce_body(p_ref, o_ref):
    o_ref[...] = jnp.sum(p_ref[...], axis=1, keepdims=True)


def _tc_reduce16(p):
    blk = 2000
    return pl.pallas_call(
        _reduce_body,
        grid=(E // blk,),
        in_specs=[pl.BlockSpec((blk, 16), lambda i: (i, 0))],
        out_specs=pl.BlockSpec((blk, 1), lambda i: (i, 0)),
        out_shape=jax.ShapeDtypeStruct((E, 1), jnp.float32),
    )(p)


@jax.jit
def kernel(x, edge_index0, edge_index1, score_edge_index, W1, b1, W2, b2):
    idx_all = jnp.stack([edge_index0, edge_index1]).reshape(2, 2, _NS, _HPW)
    pad = jnp.broadcast_to(
        (jnp.arange(_HPWP - _HPW, dtype=jnp.int32) % 200) + N,
        (2, 2, _NS, _HPWP - _HPW))
    idx_all = jnp.concatenate([idx_all, pad], axis=-1).reshape(
        2, 2, _NS, _HPWP // D, D)
    hist_parts = _hist(idx_all).reshape(4, _NS, _HROWS * D)
    hist = _tc_sum_tiles(hist_parts)
    deg_out0 = hist[0, :N, None]
    deg_in0 = hist[1, :N, None]
    deg_out1 = hist[2, :N, None]
    deg_in1 = hist[3, :N, None]

    src0 = edge_index0[0].reshape(_NC, _NS, _CH, _CW)
    dst0 = edge_index0[1].reshape(_NC, _NS, _CH, _CW)
    src1 = edge_index1[0].reshape(_NC, _NS, _CH, _CW)
    dst1 = edge_index1[1].reshape(_NC, _NS, _CH, _CW)

    hs1 = _tc_scale_matmul(x, W1, deg_out0)
    p1 = _conv(hs1, src0, dst0)
    hs2 = _tc_post_matmul(p1[0, :N], p1[1, :N], deg_in0, b1[None, :],
                          W2, deg_out1)
    p2 = _conv(hs2, src1, dst1)
    h2 = _tc_post(p2[0, :N], p2[1, :N], deg_in1, b2[None, :])

    u = score_edge_index[0].reshape(_NC, _NS, _CH, _CW)
    v = score_edge_index[1].reshape(_NC, _NS, _CH, _CW)
    partial = _score(h2, u, v).reshape(E, 16)
    return _tc_reduce16(partial)


# speedup vs baseline: 5.0146x; 1.1159x over previous
import functools

import jax
import jax.numpy as jnp
from jax import lax
from jax.experimental import pallas as pl
from jax.experimental.pallas import tpu as pltpu
from jax.experimental.pallas import tpu_sc as plsc

N = 10000
E = 320000
D = 128

_NC = 2
_NS = 16
_NW = _NC * _NS
_EPW = E // _NW
_CW = 125
_CH = _EPW // _CW
_NPAD = 10112
_RPT = _NPAD // _NS
_HPW = E // _NS
_HCH = _HPW // _CW

_BLK = 1000


def _scale_matmul_body(x_ref, w_ref, deg_ref, o_ref):
    h = jnp.dot(x_ref[...], w_ref[...], preferred_element_type=jnp.float32)
    norm = jax.lax.rsqrt(jnp.clip(deg_ref[...], 1.0, None))
    o_ref[...] = h * norm


def _tc_scale_matmul(x, W, deg):
    return pl.pallas_call(
        _scale_matmul_body,
        grid=(N // _BLK,),
        in_specs=[
            pl.BlockSpec((_BLK, D), lambda i: (i, 0)),
            pl.BlockSpec((D, D), lambda i: (0, 0)),
            pl.BlockSpec((_BLK, 1), lambda i: (i, 0)),
        ],
        out_specs=pl.BlockSpec((_BLK, D), lambda i: (i, 0)),
        out_shape=jax.ShapeDtypeStruct((N, D), jnp.float32),
    )(x, W, deg)


def _post_body(p0_ref, p1_ref, deg_ref, b_ref, o_ref):
    norm = jax.lax.rsqrt(jnp.clip(deg_ref[...], 1.0, None))
    o_ref[...] = jnp.maximum((p0_ref[...] + p1_ref[...]) * norm + b_ref[...], 0.0)


def _tc_post(p0, p1, deg, b):
    return pl.pallas_call(
        _post_body,
        grid=(N // _BLK,),
        in_specs=[
            pl.BlockSpec((_BLK, D), lambda i: (i, 0)),
            pl.BlockSpec((_BLK, D), lambda i: (i, 0)),
            pl.BlockSpec((_BLK, 1), lambda i: (i, 0)),
            pl.BlockSpec((1, D), lambda i: (0, 0)),
        ],
        out_specs=pl.BlockSpec((_BLK, D), lambda i: (i, 0)),
        out_shape=jax.ShapeDtypeStruct((N, D), jnp.float32),
    )(p0, p1, deg, b)


def _post_matmul_body(p0_ref, p1_ref, degi_ref, b_ref, w_ref, dego_ref, o_ref):
    normi = jax.lax.rsqrt(jnp.clip(degi_ref[...], 1.0, None))
    h = jnp.maximum((p0_ref[...] + p1_ref[...]) * normi + b_ref[...], 0.0)
    normo = jax.lax.rsqrt(jnp.clip(dego_ref[...], 1.0, None))
    o_ref[...] = jnp.dot(h, w_ref[...], preferred_element_type=jnp.float32) * normo


def _tc_post_matmul(p0, p1, deg_in, b, W, deg_out):
    return pl.pallas_call(
        _post_matmul_body,
        grid=(N // _BLK,),
        in_specs=[
            pl.BlockSpec((_BLK, D), lambda i: (i, 0)),
            pl.BlockSpec((_BLK, D), lambda i: (i, 0)),
            pl.BlockSpec((_BLK, 1), lambda i: (i, 0)),
            pl.BlockSpec((1, D), lambda i: (0, 0)),
            pl.BlockSpec((D, D), lambda i: (0, 0)),
            pl.BlockSpec((_BLK, 1), lambda i: (i, 0)),
        ],
        out_specs=pl.BlockSpec((_BLK, D), lambda i: (i, 0)),
        out_shape=jax.ShapeDtypeStruct((N, D), jnp.float32),
    )(p0, p1, deg_in, b, W, deg_out)


_MESH = plsc.VectorSubcoreMesh(core_axis_name="c", subcore_axis_name="s")


def _zero_rows(buf, nrows):
    ncol = buf.shape[1] // 16
    zeros = jnp.zeros((16,), jnp.float32)

    def row(i, carry):
        for k in range(ncol):
            buf[i, pl.ds(16 * k, 16)] = zeros
        return carry

    lax.fori_loop(0, nrows, row, 0)


def _zero_acc(zb, acc, base):
    _zero_rows(zb, 8)

    def blk(i, carry):
        pltpu.sync_copy(zb, acc.at[pl.ds(base + i * 8, 8)])
        return carry

    lax.fori_loop(0, _RPT // 8, blk, 0)


def _conv_body(hs_hbm, src_hbm, dst_hbm, out_hbm,
               idx_s, idx_d, rows, zbuf, acc, sem):
    c = lax.axis_index("c")
    s = lax.axis_index("s")
    pltpu.sync_copy(src_hbm.at[c, s], idx_s)
    pltpu.sync_copy(dst_hbm.at[c, s], idx_d)
    _zero_acc(zbuf, acc, s * _RPT)
    plsc.subcore_barrier()

    def chunk(j, carry):
        pltpu.async_copy(hs_hbm.at[idx_s.at[j]], rows, sem).wait()
        pltpu.sync_copy(rows, acc.at[idx_d.at[j]], add=True)
        return carry

    lax.fori_loop(0, _CH, chunk, 0)
    plsc.subcore_barrier()
    pltpu.sync_copy(acc.at[pl.ds(s * _RPT, _RPT)],
                    out_hbm.at[c, pl.ds(s * _RPT, _RPT)])


_conv = pl.kernel(
    _conv_body,
    out_type=jax.ShapeDtypeStruct((_NC, _NPAD, D), jnp.float32),
    mesh=_MESH,
    scratch_types=[
        pltpu.VMEM((_CH, _CW), jnp.int32),
        pltpu.VMEM((_CH, _CW), jnp.int32),
        pltpu.VMEM((_CW, D), jnp.float32),
        pltpu.VMEM((8, D), jnp.float32),
        pltpu.VMEM_SHARED((_NPAD, D), jnp.float32),
        pltpu.SemaphoreType.DMA,
    ],
)


_HROWS = 80
_HPWP = 20096


def _hist_one(idxbuf, hist):
    iota = lax.iota(jnp.int32, 16)
    pib = jax.lax.GatherScatterMode.PROMISE_IN_BOUNDS

    def step(k, carry):
        for g in range(8):
            idx = idxbuf[k, pl.ds(g * 16, 16)]
            sk, _ = plsc.sort_key_val(idx, idx)
            prev = sk.at[jnp.maximum(iota - 1, 0)].get(mode=pib)
            nxt = sk.at[jnp.minimum(iota + 1, 15)].get(mode=pib)
            head = (iota == 0) | (sk != prev)
            last = (iota == 15) | (sk != nxt)
            start = plsc.cummax(jnp.where(head, iota, 0))
            cnt = (iota - start + 1).astype(jnp.float32)
            plsc.addupdate_scatter(
                hist,
                [lax.shift_right_logical(sk, 7), lax.bitwise_and(sk, 127)],
                cnt, mask=last)
        return carry

    lax.fori_loop(0, _HPWP // D, step, 0)


def _hist_body(idx_hbm, out_hbm, idx_s, idx_d, hist_s, hist_d):
    c = lax.axis_index("c")
    s = lax.axis_index("s")
    pltpu.sync_copy(idx_hbm.at[c, 0, s], idx_s)
    pltpu.sync_copy(idx_hbm.at[c, 1, s], idx_d)
    _zero_rows(hist_s, _HROWS)
    _zero_rows(hist_d, _HROWS)
    _hist_one(idx_s, hist_s)
    _hist_one(idx_d, hist_d)
    pltpu.sync_copy(hist_s, out_hbm.at[c, 0, s])
    pltpu.sync_copy(hist_d, out_hbm.at[c, 1, s])


_hist = pl.kernel(
    _hist_body,
    out_type=jax.ShapeDtypeStruct((_NC, 2, _NS, _HROWS, D), jnp.float32),
    mesh=_MESH,
    scratch_types=[
        pltpu.VMEM((_HPWP // D, D), jnp.int32),
        pltpu.VMEM((_HPWP // D, D), jnp.int32),
        pltpu.VMEM((_HROWS, D), jnp.float32),
        pltpu.VMEM((_HROWS, D), jnp.float32),
    ],
    compiler_params=pltpu.CompilerParams(needs_layout_passes=False),
)


def _sum_tiles_body(p_ref, o_ref):
    o_ref[...] = jnp.sum(p_ref[...], axis=1)


def _tc_sum_tiles(p):
    hb = _HROWS * D
    return pl.pallas_call(
        _sum_tiles_body,
        out_shape=jax.ShapeDtypeStruct((4, hb), jnp.float32),
    )(p)


def _score_body(h_hbm, u_hbm, v_hbm, out_hbm,
                idx_u, idx_v, ubuf0, vbuf0, ubuf1, vbuf1, obuf,
                semu0, semv0, semu1, semv1):
    c = lax.axis_index("c")
    s = lax.axis_index("s")
    pltpu.sync_copy(u_hbm.at[c, s], idx_u)
    pltpu.sync_copy(v_hbm.at[c, s], idx_v)

    def start(j, ub, vb, su, sv):
        pltpu.async_copy(h_hbm.at[idx_u.at[j]], ub, su)
        pltpu.async_copy(h_hbm.at[idx_v.at[j]], vb, sv)

    def wait(ub, vb, su, sv):
        pltpu.make_async_copy(h_hbm.at[idx_u.at[0]], ub, su).wait()
        pltpu.make_async_copy(h_hbm.at[idx_v.at[0]], vb, sv).wait()

    def compute(j, ub, vb):
        def row(r, carry2):
            acc = ub[r, pl.ds(0, 16)] * vb[r, pl.ds(0, 16)]
            for k in range(1, D // 16):
                acc = acc + ub[r, pl.ds(16 * k, 16)] * vb[r, pl.ds(16 * k, 16)]
            obuf[r, pl.ds(0, 16)] = acc
            return carry2

        lax.fori_loop(0, _CW, row, 0)
        pltpu.sync_copy(obuf, out_hbm.at[c, s, j])

    start(0, ubuf0, vbuf0, semu0, semv0)

    def pair(t, carry):
        start(2 * t + 1, ubuf1, vbuf1, semu1, semv1)
        wait(ubuf0, vbuf0, semu0, semv0)
        compute(2 * t, ubuf0, vbuf0)

        @pl.when(t < _CH // 2 - 1)
        def _():
            start(2 * t + 2, ubuf0, vbuf0, semu0, semv0)

        wait(ubuf1, vbuf1, semu1, semv1)
        compute(2 * t + 1, ubuf1, vbuf1)
        return carry

    lax.fori_loop(0, _CH // 2, pair, 0)


_score = pl.kernel(
    _score_body,
    out_type=jax.ShapeDtypeStruct((_NC, _NS, _CH, _CW, 16), jnp.float32),
    mesh=_MESH,
    scratch_types=[
        pltpu.VMEM((_CH, _CW), jnp.int32),
        pltpu.VMEM((_CH, _CW), jnp.int32),
        pltpu.VMEM((_CW, D), jnp.float32),
        pltpu.VMEM((_CW, D), jnp.float32),
        pltpu.VMEM((_CW, D), jnp.float32),
        pltpu.VMEM((_CW, D), jnp.float32),
        pltpu.VMEM((_CW, 16), jnp.float32),
        pltpu.SemaphoreType.DMA,
        pltpu.SemaphoreType.DMA,
        pltpu.SemaphoreType.DMA,
        pltpu.SemaphoreType.DMA,
    ],
)


def _reduce_body(p_ref, o_ref):
    o_ref[...] = jnp.sum(p_ref[...], axis=1, keepdims=True)


def _tc_reduce16(p):
    blk = 2000
    return pl.pallas_call(
        _reduce_body,
        grid=(E // blk,),
        in_specs=[pl.BlockSpec((blk, 16), lambda i: (i, 0))],
        out_specs=pl.BlockSpec((blk, 1), lambda i: (i, 0)),
        out_shape=jax.ShapeDtypeStruct((E, 1), jnp.float32),
    )(p)


@jax.jit
def kernel(x, edge_index0, edge_index1, score_edge_index, W1, b1, W2, b2):
    idx_all = jnp.stack([edge_index0, edge_index1]).reshape(2, 2, _NS, _HPW)
    pad = jnp.broadcast_to(
        (jnp.arange(_HPWP - _HPW, dtype=jnp.int32) % 200) + N,
        (2, 2, _NS, _HPWP - _HPW))
    idx_all = jnp.concatenate([idx_all, pad], axis=-1).reshape(
        2, 2, _NS, _HPWP // D, D)
    hist_parts = _hist(idx_all).reshape(4, _NS, _HROWS * D)
    hist = _tc_sum_tiles(hist_parts)
    deg_out0 = hist[0, :N, None]
    deg_in0 = hist[1, :N, None]
    deg_out1 = hist[2, :N, None]
    deg_in1 = hist[3, :N, None]

    src0 = edge_index0[0].reshape(_NC, _NS, _CH, _CW)
    dst0 = edge_index0[1].reshape(_NC, _NS, _CH, _CW)
    src1 = edge_index1[0].reshape(_NC, _NS, _CH, _CW)
    dst1 = edge_index1[1].reshape(_NC, _NS, _CH, _CW)

    hs1 = _tc_scale_matmul(x, W1, deg_out0)
    p1 = _conv(hs1, src0, dst0)
    hs2 = _tc_post_matmul(p1[0, :N], p1[1, :N], deg_in0, b1[None, :],
                          W2, deg_out1)
    p2 = _conv(hs2, src1, dst1)
    h2 = _tc_post(p2[0, :N], p2[1, :N], deg_in1, b2[None, :])

    u = score_edge_index[0].reshape(_NC, _NS, _CH, _CW)
    v = score_edge_index[1].reshape(_NC, _NS, _CH, _CW)
    partial = _score(h2, u, v).reshape(E, 16)
    return _tc_reduce16(partial)


# speedup vs baseline: 5.7026x; 1.1372x over previous
import functools

import jax
import jax.numpy as jnp
from jax import lax
from jax.experimental import pallas as pl
from jax.experimental.pallas import tpu as pltpu
from jax.experimental.pallas import tpu_sc as plsc

N = 10000
E = 320000
D = 128

_NC = 2
_NS = 16
_NW = _NC * _NS
_EPW = E // _NW
_CW = 125
_CH = _EPW // _CW
_NPAD = 10112
_RPT = _NPAD // _NS
_HPW = E // _NS
_HCH = _HPW // _CW

_BLK = 1000


def _scale_matmul_body(x_ref, w_ref, deg_ref, o_ref):
    h = jnp.dot(x_ref[...], w_ref[...], preferred_element_type=jnp.float32)
    norm = jax.lax.rsqrt(jnp.clip(deg_ref[...], 1.0, None))
    o_ref[...] = h * norm


def _tc_scale_matmul(x, W, deg):
    return pl.pallas_call(
        _scale_matmul_body,
        grid=(N // _BLK,),
        in_specs=[
            pl.BlockSpec((_BLK, D), lambda i: (i, 0)),
            pl.BlockSpec((D, D), lambda i: (0, 0)),
            pl.BlockSpec((_BLK, 1), lambda i: (i, 0)),
        ],
        out_specs=pl.BlockSpec((_BLK, D), lambda i: (i, 0)),
        out_shape=jax.ShapeDtypeStruct((N, D), jnp.float32),
    )(x, W, deg)


def _post_body(p0_ref, p1_ref, deg_ref, b_ref, o_ref):
    norm = jax.lax.rsqrt(jnp.clip(deg_ref[...], 1.0, None))
    o_ref[...] = jnp.maximum((p0_ref[...] + p1_ref[...]) * norm + b_ref[...], 0.0)


def _tc_post(p0, p1, deg, b):
    return pl.pallas_call(
        _post_body,
        grid=(N // _BLK,),
        in_specs=[
            pl.BlockSpec((_BLK, D), lambda i: (i, 0)),
            pl.BlockSpec((_BLK, D), lambda i: (i, 0)),
            pl.BlockSpec((_BLK, 1), lambda i: (i, 0)),
            pl.BlockSpec((1, D), lambda i: (0, 0)),
        ],
        out_specs=pl.BlockSpec((_BLK, D), lambda i: (i, 0)),
        out_shape=jax.ShapeDtypeStruct((N, D), jnp.float32),
    )(p0, p1, deg, b)


def _post_matmul_body(p0_ref, p1_ref, degi_ref, b_ref, w_ref, dego_ref, o_ref):
    normi = jax.lax.rsqrt(jnp.clip(degi_ref[...], 1.0, None))
    h = jnp.maximum((p0_ref[...] + p1_ref[...]) * normi + b_ref[...], 0.0)
    normo = jax.lax.rsqrt(jnp.clip(dego_ref[...], 1.0, None))
    o_ref[...] = jnp.dot(h, w_ref[...], preferred_element_type=jnp.float32) * normo


def _tc_post_matmul(p0, p1, deg_in, b, W, deg_out):
    return pl.pallas_call(
        _post_matmul_body,
        grid=(N // _BLK,),
        in_specs=[
            pl.BlockSpec((_BLK, D), lambda i: (i, 0)),
            pl.BlockSpec((_BLK, D), lambda i: (i, 0)),
            pl.BlockSpec((_BLK, 1), lambda i: (i, 0)),
            pl.BlockSpec((1, D), lambda i: (0, 0)),
            pl.BlockSpec((D, D), lambda i: (0, 0)),
            pl.BlockSpec((_BLK, 1), lambda i: (i, 0)),
        ],
        out_specs=pl.BlockSpec((_BLK, D), lambda i: (i, 0)),
        out_shape=jax.ShapeDtypeStruct((N, D), jnp.float32),
    )(p0, p1, deg_in, b, W, deg_out)


_MESH = plsc.VectorSubcoreMesh(core_axis_name="c", subcore_axis_name="s")


def _zero_rows(buf, nrows):
    ncol = buf.shape[1] // 16
    zeros = jnp.zeros((16,), jnp.float32)

    def row(i, carry):
        for k in range(ncol):
            buf[i, pl.ds(16 * k, 16)] = zeros
        return carry

    lax.fori_loop(0, nrows, row, 0)


def _zero_acc(zb, acc, base):
    _zero_rows(zb, 8)

    def blk(i, carry):
        pltpu.sync_copy(zb, acc.at[pl.ds(base + i * 8, 8)])
        return carry

    lax.fori_loop(0, _RPT // 8, blk, 0)


_CHH = _CH // 2


def _conv_body(hs_hbm, src_hbm, dst_hbm, out_hbm,
               idx_s, idx_d, rows0, rows1, zbuf, acc, sem0, sem1):
    c = lax.axis_index("c")
    s = lax.axis_index("s")
    _zero_acc(zbuf, acc, s * _RPT)
    plsc.subcore_barrier()

    def start(j, rb, sem):
        pltpu.async_copy(hs_hbm.at[idx_s.at[j]], rb, sem)

    def wait(rb, sem):
        pltpu.make_async_copy(hs_hbm.at[idx_s.at[0]], rb, sem).wait()

    def scat(j, rb):
        pltpu.sync_copy(rb, acc.at[idx_d.at[j]], add=True)

    for h in range(2):
        pltpu.sync_copy(src_hbm.at[c, s, pl.ds(h * _CHH, _CHH)], idx_s)
        pltpu.sync_copy(dst_hbm.at[c, s, pl.ds(h * _CHH, _CHH)], idx_d)
        start(0, rows0, sem0)

        def pair(t, carry):
            start(2 * t + 1, rows1, sem1)
            wait(rows0, sem0)
            scat(2 * t, rows0)

            @pl.when(t < _CHH // 2 - 1)
            def _():
                start(2 * t + 2, rows0, sem0)

            wait(rows1, sem1)
            scat(2 * t + 1, rows1)
            return carry

        lax.fori_loop(0, _CHH // 2, pair, 0)

    plsc.subcore_barrier()
    pltpu.sync_copy(acc.at[pl.ds(s * _RPT, _RPT)],
                    out_hbm.at[c, pl.ds(s * _RPT, _RPT)])


_conv = pl.kernel(
    _conv_body,
    out_type=jax.ShapeDtypeStruct((_NC, _NPAD, D), jnp.float32),
    mesh=_MESH,
    scratch_types=[
        pltpu.VMEM((_CHH, _CW), jnp.int32),
        pltpu.VMEM((_CHH, _CW), jnp.int32),
        pltpu.VMEM((_CW, D), jnp.float32),
        pltpu.VMEM((_CW, D), jnp.float32),
        pltpu.VMEM((8, D), jnp.float32),
        pltpu.VMEM_SHARED((_NPAD, D), jnp.float32),
        pltpu.SemaphoreType.DMA,
        pltpu.SemaphoreType.DMA,
    ],
)


_HROWS = 80
_HPWP = 20096


def _hist_one(idxbuf, hist):
    iota = lax.iota(jnp.int32, 16)
    pib = jax.lax.GatherScatterMode.PROMISE_IN_BOUNDS

    def step(k, carry):
        for g in range(8):
            idx = idxbuf[k, pl.ds(g * 16, 16)]
            sk, _ = plsc.sort_key_val(idx, idx)
            prev = sk.at[jnp.maximum(iota - 1, 0)].get(mode=pib)
            nxt = sk.at[jnp.minimum(iota + 1, 15)].get(mode=pib)
            head = (iota == 0) | (sk != prev)
            last = (iota == 15) | (sk != nxt)
            start = plsc.cummax(jnp.where(head, iota, 0))
            cnt = (iota - start + 1).astype(jnp.float32)
            plsc.addupdate_scatter(
                hist,
                [lax.shift_right_logical(sk, 7), lax.bitwise_and(sk, 127)],
                cnt, mask=last)
        return carry

    lax.fori_loop(0, _HPWP // D, step, 0)


def _hist_body(idx_hbm, out_hbm, idx_s, idx_d, hist_s, hist_d):
    c = lax.axis_index("c")
    s = lax.axis_index("s")
    pltpu.sync_copy(idx_hbm.at[c, 0, s], idx_s)
    pltpu.sync_copy(idx_hbm.at[c, 1, s], idx_d)
    _zero_rows(hist_s, _HROWS)
    _zero_rows(hist_d, _HROWS)
    _hist_one(idx_s, hist_s)
    _hist_one(idx_d, hist_d)
    pltpu.sync_copy(hist_s, out_hbm.at[c, 0, s])
    pltpu.sync_copy(hist_d, out_hbm.at[c, 1, s])


_hist = pl.kernel(
    _hist_body,
    out_type=jax.ShapeDtypeStruct((_NC, 2, _NS, _HROWS, D), jnp.float32),
    mesh=_MESH,
    scratch_types=[
        pltpu.VMEM((_HPWP // D, D), jnp.int32),
        pltpu.VMEM((_HPWP // D, D), jnp.int32),
        pltpu.VMEM((_HROWS, D), jnp.float32),
        pltpu.VMEM((_HROWS, D), jnp.float32),
    ],
    compiler_params=pltpu.CompilerParams(needs_layout_passes=False),
)


def _sum_tiles_body(p_ref, o_ref):
    o_ref[...] = jnp.sum(p_ref[...], axis=1)


def _tc_sum_tiles(p):
    hb = _HROWS * D
    return pl.pallas_call(
        _sum_tiles_body,
        out_shape=jax.ShapeDtypeStruct((4, hb), jnp.float32),
    )(p)


def _score_body(h_hbm, u_hbm, v_hbm, out_hbm,
                idx_u, idx_v, ubuf0, vbuf0, ubuf1, vbuf1, obuf,
                semu0, semv0, semu1, semv1):
    c = lax.axis_index("c")
    s = lax.axis_index("s")
    pltpu.sync_copy(u_hbm.at[c, s], idx_u)
    pltpu.sync_copy(v_hbm.at[c, s], idx_v)

    def start(j, ub, vb, su, sv):
        pltpu.async_copy(h_hbm.at[idx_u.at[j]], ub, su)
        pltpu.async_copy(h_hbm.at[idx_v.at[j]], vb, sv)

    def wait(ub, vb, su, sv):
        pltpu.make_async_copy(h_hbm.at[idx_u.at[0]], ub, su).wait()
        pltpu.make_async_copy(h_hbm.at[idx_v.at[0]], vb, sv).wait()

    def compute(j, ub, vb):
        def row(r, carry2):
            acc = ub[r, pl.ds(0, 16)] * vb[r, pl.ds(0, 16)]
            for k in range(1, D // 16):
                acc = acc + ub[r, pl.ds(16 * k, 16)] * vb[r, pl.ds(16 * k, 16)]
            obuf[r, pl.ds(0, 16)] = acc
            return carry2

        lax.fori_loop(0, _CW, row, 0)
        pltpu.sync_copy(obuf, out_hbm.at[c, s, j])

    start(0, ubuf0, vbuf0, semu0, semv0)

    def pair(t, carry):
        start(2 * t + 1, ubuf1, vbuf1, semu1, semv1)
        wait(ubuf0, vbuf0, semu0, semv0)
        compute(2 * t, ubuf0, vbuf0)

        @pl.when(t < _CH // 2 - 1)
        def _():
            start(2 * t + 2, ubuf0, vbuf0, semu0, semv0)

        wait(ubuf1, vbuf1, semu1, semv1)
        compute(2 * t + 1, ubuf1, vbuf1)
        return carry

    lax.fori_loop(0, _CH // 2, pair, 0)


_score = pl.kernel(
    _score_body,
    out_type=jax.ShapeDtypeStruct((_NC, _NS, _CH, _CW, 16), jnp.float32),
    mesh=_MESH,
    scratch_types=[
        pltpu.VMEM((_CH, _CW), jnp.int32),
        pltpu.VMEM((_CH, _CW), jnp.int32),
        pltpu.VMEM((_CW, D), jnp.float32),
        pltpu.VMEM((_CW, D), jnp.float32),
        pltpu.VMEM((_CW, D), jnp.float32),
        pltpu.VMEM((_CW, D), jnp.float32),
        pltpu.VMEM((_CW, 16), jnp.float32),
        pltpu.SemaphoreType.DMA,
        pltpu.SemaphoreType.DMA,
        pltpu.SemaphoreType.DMA,
        pltpu.SemaphoreType.DMA,
    ],
)


def _reduce_body(p_ref, o_ref):
    o_ref[...] = jnp.sum(p_ref[...], axis=1, keepdims=True)


def _tc_reduce16(p):
    blk = 2000
    return pl.pallas_call(
        _reduce_body,
        grid=(E // blk,),
        in_specs=[pl.BlockSpec((blk, 16), lambda i: (i, 0))],
        out_specs=pl.BlockSpec((blk, 1), lambda i: (i, 0)),
        out_shape=jax.ShapeDtypeStruct((E, 1), jnp.float32),
    )(p)


@jax.jit
def kernel(x, edge_index0, edge_index1, score_edge_index, W1, b1, W2, b2):
    idx_all = jnp.stack([edge_index0, edge_index1]).reshape(2, 2, _NS, _HPW)
    pad = jnp.broadcast_to(
        (jnp.arange(_HPWP - _HPW, dtype=jnp.int32) % 200) + N,
        (2, 2, _NS, _HPWP - _HPW))
    idx_all = jnp.concatenate([idx_all, pad], axis=-1).reshape(
        2, 2, _NS, _HPWP // D, D)
    hist_parts = _hist(idx_all).reshape(4, _NS, _HROWS * D)
    hist = _tc_sum_tiles(hist_parts)
    deg_out0 = hist[0, :N, None]
    deg_in0 = hist[1, :N, None]
    deg_out1 = hist[2, :N, None]
    deg_in1 = hist[3, :N, None]

    src0 = edge_index0[0].reshape(_NC, _NS, _CH, _CW)
    dst0 = edge_index0[1].reshape(_NC, _NS, _CH, _CW)
    src1 = edge_index1[0].reshape(_NC, _NS, _CH, _CW)
    dst1 = edge_index1[1].reshape(_NC, _NS, _CH, _CW)

    hs1 = _tc_scale_matmul(x, W1, deg_out0)
    p1 = _conv(hs1, src0, dst0)
    hs2 = _tc_post_matmul(p1[0, :N], p1[1, :N], deg_in0, b1[None, :],
                          W2, deg_out1)
    p2 = _conv(hs2, src1, dst1)
    h2 = _tc_post(p2[0, :N], p2[1, :N], deg_in1, b2[None, :])

    u = score_edge_index[0].reshape(_NC, _NS, _CH, _CW)
    v = score_edge_index[1].reshape(_NC, _NS, _CH, _CW)
    partial = _score(h2, u, v).reshape(E, 16)
    return _tc_reduce16(partial)


# speedup vs baseline: 5.7868x; 1.0148x over previous
import functools

import jax
import jax.numpy as jnp
from jax import lax
from jax.experimental import pallas as pl
from jax.experimental.pallas import tpu as pltpu
from jax.experimental.pallas import tpu_sc as plsc

N = 10000
E = 320000
D = 128

_NC = 2
_NS = 16
_NW = _NC * _NS
_EPW = E // _NW
_CW = 125
_CH = _EPW // _CW
_NPAD = 10112
_RPT = _NPAD // _NS
_HPW = E // _NS
_HCH = _HPW // _CW

_BLK = 1000


def _scale_matmul_body(x_ref, w_ref, deg_ref, o_ref):
    h = jnp.dot(x_ref[...], w_ref[...], preferred_element_type=jnp.float32)
    norm = jax.lax.rsqrt(jnp.clip(deg_ref[...], 1.0, None))
    o_ref[...] = h * norm


def _tc_scale_matmul(x, W, deg):
    return pl.pallas_call(
        _scale_matmul_body,
        grid=(N // _BLK,),
        in_specs=[
            pl.BlockSpec((_BLK, D), lambda i: (i, 0)),
            pl.BlockSpec((D, D), lambda i: (0, 0)),
            pl.BlockSpec((_BLK, 1), lambda i: (i, 0)),
        ],
        out_specs=pl.BlockSpec((_BLK, D), lambda i: (i, 0)),
        out_shape=jax.ShapeDtypeStruct((N, D), jnp.float32),
    )(x, W, deg)


def _post_body(p0_ref, p1_ref, deg_ref, b_ref, o_ref):
    norm = jax.lax.rsqrt(jnp.clip(deg_ref[...], 1.0, None))
    o_ref[...] = jnp.maximum(
        (p0_ref[0] + p1_ref[0]) * norm + b_ref[...], 0.0)


def _tc_post(p, deg, b):
    return pl.pallas_call(
        _post_body,
        grid=(N // _BLK,),
        in_specs=[
            pl.BlockSpec((1, _BLK, D), lambda i: (0, i, 0)),
            pl.BlockSpec((1, _BLK, D), lambda i: (1, i, 0)),
            pl.BlockSpec((_BLK, 1), lambda i: (i, 0)),
            pl.BlockSpec((1, D), lambda i: (0, 0)),
        ],
        out_specs=pl.BlockSpec((_BLK, D), lambda i: (i, 0)),
        out_shape=jax.ShapeDtypeStruct((N, D), jnp.float32),
    )(p, p, deg, b)


def _post_matmul_body(p0_ref, p1_ref, degi_ref, b_ref, w_ref, dego_ref, o_ref):
    normi = jax.lax.rsqrt(jnp.clip(degi_ref[...], 1.0, None))
    h = jnp.maximum((p0_ref[0] + p1_ref[0]) * normi + b_ref[...], 0.0)
    normo = jax.lax.rsqrt(jnp.clip(dego_ref[...], 1.0, None))
    o_ref[...] = jnp.dot(h, w_ref[...], preferred_element_type=jnp.float32) * normo


def _tc_post_matmul(p, deg_in, b, W, deg_out):
    return pl.pallas_call(
        _post_matmul_body,
        grid=(N // _BLK,),
        in_specs=[
            pl.BlockSpec((1, _BLK, D), lambda i: (0, i, 0)),
            pl.BlockSpec((1, _BLK, D), lambda i: (1, i, 0)),
            pl.BlockSpec((_BLK, 1), lambda i: (i, 0)),
            pl.BlockSpec((1, D), lambda i: (0, 0)),
            pl.BlockSpec((D, D), lambda i: (0, 0)),
            pl.BlockSpec((_BLK, 1), lambda i: (i, 0)),
        ],
        out_specs=pl.BlockSpec((_BLK, D), lambda i: (i, 0)),
        out_shape=jax.ShapeDtypeStruct((N, D), jnp.float32),
    )(p, p, deg_in, b, W, deg_out)


_MESH = plsc.VectorSubcoreMesh(core_axis_name="c", subcore_axis_name="s")


def _zero_rows(buf, nrows):
    ncol = buf.shape[1] // 16
    zeros = jnp.zeros((16,), jnp.float32)

    def row(i, carry):
        for k in range(ncol):
            buf[i, pl.ds(16 * k, 16)] = zeros
        return carry

    lax.fori_loop(0, nrows, row, 0)


def _zero_acc(zb, acc, base):
    _zero_rows(zb, 8)

    def blk(i, carry):
        pltpu.sync_copy(zb, acc.at[pl.ds(base + i * 8, 8)])
        return carry

    lax.fori_loop(0, _RPT // 8, blk, 0)


_CHH = _CH // 2


def _conv_body(hs_hbm, src_hbm, dst_hbm, out_hbm,
               idx_s, idx_d, rows0, rows1, zbuf, acc, sem0, sem1):
    c = lax.axis_index("c")
    s = lax.axis_index("s")
    _zero_acc(zbuf, acc, s * _RPT)
    plsc.subcore_barrier()

    def start(j, rb, sem):
        pltpu.async_copy(hs_hbm.at[idx_s.at[j]], rb, sem)

    def wait(rb, sem):
        pltpu.make_async_copy(hs_hbm.at[idx_s.at[0]], rb, sem).wait()

    def scat(j, rb):
        pltpu.sync_copy(rb, acc.at[idx_d.at[j]], add=True)

    for h in range(2):
        pltpu.sync_copy(src_hbm.at[c, s, pl.ds(h * _CHH, _CHH)], idx_s)
        pltpu.sync_copy(dst_hbm.at[c, s, pl.ds(h * _CHH, _CHH)], idx_d)
        start(0, rows0, sem0)

        def pair(t, carry):
            start(2 * t + 1, rows1, sem1)
            wait(rows0, sem0)
            scat(2 * t, rows0)

            @pl.when(t < _CHH // 2 - 1)
            def _():
                start(2 * t + 2, rows0, sem0)

            wait(rows1, sem1)
            scat(2 * t + 1, rows1)
            return carry

        lax.fori_loop(0, _CHH // 2, pair, 0)

    plsc.subcore_barrier()
    pltpu.sync_copy(acc.at[pl.ds(s * _RPT, _RPT)],
                    out_hbm.at[c, pl.ds(s * _RPT, _RPT)])


_conv = pl.kernel(
    _conv_body,
    out_type=jax.ShapeDtypeStruct((_NC, _NPAD, D), jnp.float32),
    mesh=_MESH,
    scratch_types=[
        pltpu.VMEM((_CHH, _CW), jnp.int32),
        pltpu.VMEM((_CHH, _CW), jnp.int32),
        pltpu.VMEM((_CW, D), jnp.float32),
        pltpu.VMEM((_CW, D), jnp.float32),
        pltpu.VMEM((8, D), jnp.float32),
        pltpu.VMEM_SHARED((_NPAD, D), jnp.float32),
        pltpu.SemaphoreType.DMA,
        pltpu.SemaphoreType.DMA,
    ],
)


_HROWS = 80
_HPWP = 20096


def _hist_one(idxbuf, hist):
    iota = lax.iota(jnp.int32, 16)
    pib = jax.lax.GatherScatterMode.PROMISE_IN_BOUNDS

    def step(k, carry):
        for g in range(8):
            idx = idxbuf[k, pl.ds(g * 16, 16)]
            sk, _ = plsc.sort_key_val(idx, idx)
            prev = sk.at[jnp.maximum(iota - 1, 0)].get(mode=pib)
            nxt = sk.at[jnp.minimum(iota + 1, 15)].get(mode=pib)
            head = (iota == 0) | (sk != prev)
            last = (iota == 15) | (sk != nxt)
            start = plsc.cummax(jnp.where(head, iota, 0))
            cnt = (iota - start + 1).astype(jnp.float32)
            plsc.addupdate_scatter(
                hist,
                [lax.shift_right_logical(sk, 7), lax.bitwise_and(sk, 127)],
                cnt, mask=last)
        return carry

    lax.fori_loop(0, _HPWP // D, step, 0)


def _hist_body(idx_hbm, out_hbm, idx_s, idx_d, hist_s, hist_d):
    c = lax.axis_index("c")
    s = lax.axis_index("s")
    pltpu.sync_copy(idx_hbm.at[c, 0, s], idx_s)
    pltpu.sync_copy(idx_hbm.at[c, 1, s], idx_d)
    _zero_rows(hist_s, _HROWS)
    _zero_rows(hist_d, _HROWS)
    _hist_one(idx_s, hist_s)
    _hist_one(idx_d, hist_d)
    pltpu.sync_copy(hist_s, out_hbm.at[c, 0, s])
    pltpu.sync_copy(hist_d, out_hbm.at[c, 1, s])


_hist = pl.kernel(
    _hist_body,
    out_type=jax.ShapeDtypeStruct((_NC, 2, _NS, _HROWS, D), jnp.float32),
    mesh=_MESH,
    scratch_types=[
        pltpu.VMEM((_HPWP // D, D), jnp.int32),
        pltpu.VMEM((_HPWP // D, D), jnp.int32),
        pltpu.VMEM((_HROWS, D), jnp.float32),
        pltpu.VMEM((_HROWS, D), jnp.float32),
    ],
    compiler_params=pltpu.CompilerParams(needs_layout_passes=False),
)


def _sum_tiles_body(p_ref, o_ref):
    o_ref[...] = jnp.sum(p_ref[...], axis=1)


def _tc_sum_tiles(p):
    hb = _HROWS * D
    return pl.pallas_call(
        _sum_tiles_body,
        out_shape=jax.ShapeDtypeStruct((4, hb), jnp.float32),
    )(p)


def _score_body(h_hbm, u_hbm, v_hbm, out_hbm,
                idx_u, idx_v, ubuf0, vbuf0, ubuf1, vbuf1, obuf,
                semu0, semv0, semu1, semv1):
    c = lax.axis_index("c")
    s = lax.axis_index("s")
    pltpu.sync_copy(u_hbm.at[c, s], idx_u)
    pltpu.sync_copy(v_hbm.at[c, s], idx_v)

    def start(j, ub, vb, su, sv):
        pltpu.async_copy(h_hbm.at[idx_u.at[j]], ub, su)
        pltpu.async_copy(h_hbm.at[idx_v.at[j]], vb, sv)

    def wait(ub, vb, su, sv):
        pltpu.make_async_copy(h_hbm.at[idx_u.at[0]], ub, su).wait()
        pltpu.make_async_copy(h_hbm.at[idx_v.at[0]], vb, sv).wait()

    def compute(j, ub, vb):
        def row(r, carry2):
            acc = ub[r, pl.ds(0, 16)] * vb[r, pl.ds(0, 16)]
            for k in range(1, D // 16):
                acc = acc + ub[r, pl.ds(16 * k, 16)] * vb[r, pl.ds(16 * k, 16)]
            obuf[r, pl.ds(0, 16)] = acc
            return carry2

        lax.fori_loop(0, _CW, row, 0)
        pltpu.sync_copy(obuf, out_hbm.at[c, s, j])

    start(0, ubuf0, vbuf0, semu0, semv0)

    def pair(t, carry):
        start(2 * t + 1, ubuf1, vbuf1, semu1, semv1)
        wait(ubuf0, vbuf0, semu0, semv0)
        compute(2 * t, ubuf0, vbuf0)

        @pl.when(t < _CH // 2 - 1)
        def _():
            start(2 * t + 2, ubuf0, vbuf0, semu0, semv0)

        wait(ubuf1, vbuf1, semu1, semv1)
        compute(2 * t + 1, ubuf1, vbuf1)
        return carry

    lax.fori_loop(0, _CH // 2, pair, 0)


_score = pl.kernel(
    _score_body,
    out_type=jax.ShapeDtypeStruct((_NC, _NS, _CH, _CW, 16), jnp.float32),
    mesh=_MESH,
    scratch_types=[
        pltpu.VMEM((_CH, _CW), jnp.int32),
        pltpu.VMEM((_CH, _CW), jnp.int32),
        pltpu.VMEM((_CW, D), jnp.float32),
        pltpu.VMEM((_CW, D), jnp.float32),
        pltpu.VMEM((_CW, D), jnp.float32),
        pltpu.VMEM((_CW, D), jnp.float32),
        pltpu.VMEM((_CW, 16), jnp.float32),
        pltpu.SemaphoreType.DMA,
        pltpu.SemaphoreType.DMA,
        pltpu.SemaphoreType.DMA,
        pltpu.SemaphoreType.DMA,
    ],
)


def _reduce_body(p_ref, o_ref):
    o_ref[...] = jnp.sum(p_ref[...], axis=1, keepdims=True)


def _tc_reduce16(p):
    blk = 2000
    return pl.pallas_call(
        _reduce_body,
        grid=(E // blk,),
        in_specs=[pl.BlockSpec((blk, 16), lambda i: (i, 0))],
        out_specs=pl.BlockSpec((blk, 1), lambda i: (i, 0)),
        out_shape=jax.ShapeDtypeStruct((E, 1), jnp.float32),
    )(p)


@jax.jit
def kernel(x, edge_index0, edge_index1, score_edge_index, W1, b1, W2, b2):
    idx_all = jnp.stack([edge_index0, edge_index1]).reshape(2, 2, _NS, _HPW)
    pad = jnp.broadcast_to(
        (jnp.arange(_HPWP - _HPW, dtype=jnp.int32) % 200) + N,
        (2, 2, _NS, _HPWP - _HPW))
    idx_all = jnp.concatenate([idx_all, pad], axis=-1).reshape(
        2, 2, _NS, _HPWP // D, D)
    hist_parts = _hist(idx_all).reshape(4, _NS, _HROWS * D)
    hist = _tc_sum_tiles(hist_parts)
    deg_out0 = hist[0, :N, None]
    deg_in0 = hist[1, :N, None]
    deg_out1 = hist[2, :N, None]
    deg_in1 = hist[3, :N, None]

    src0 = edge_index0[0].reshape(_NC, _NS, _CH, _CW)
    dst0 = edge_index0[1].reshape(_NC, _NS, _CH, _CW)
    src1 = edge_index1[0].reshape(_NC, _NS, _CH, _CW)
    dst1 = edge_index1[1].reshape(_NC, _NS, _CH, _CW)

    hs1 = _tc_scale_matmul(x, W1, deg_out0)
    p1 = _conv(hs1, src0, dst0)
    hs2 = _tc_post_matmul(p1, deg_in0, b1[None, :], W2, deg_out1)
    p2 = _conv(hs2, src1, dst1)
    h2 = _tc_post(p2, deg_in1, b2[None, :])

    u = score_edge_index[0].reshape(_NC, _NS, _CH, _CW)
    v = score_edge_index[1].reshape(_NC, _NS, _CH, _CW)
    partial = _score(h2, u, v).reshape(E, 16)
    return _tc_reduce16(partial)
